# baseline (speedup 1.0000x reference)
import functools

import jax
import jax.numpy as jnp
from jax import lax
from jax.experimental import pallas as pl
from jax.experimental.pallas import tpu as pltpu
from jax.experimental.pallas import tpu_sc as plsc

L = 16


def _softplus(x):
    t = jnp.exp(-jnp.abs(x))
    s = t / (t + 2.0)
    s2 = s * s
    atanh_s = s * (1.0 + s2 * (1.0 / 3.0 + s2 * (0.2 + s2 * (1.0 / 7.0 + s2 * (1.0 / 9.0)))))
    return jnp.maximum(x, 0.0) + 2.0 * atanh_s


def _make_sc_kernel(B, K, NI, NW, NC):
    bpw = B // NW
    nv = bpw // L
    mesh = plsc.VectorSubcoreMesh(core_axis_name="c", subcore_axis_name="s")

    @functools.partial(
        pl.kernel,
        mesh=mesh,
        compiler_params=pltpu.CompilerParams(needs_layout_passes=False),
        out_type=jax.ShapeDtypeStruct((B,), jnp.float32),
        scratch_types=[
            pltpu.VMEM((bpw,), jnp.int32),
            pltpu.VMEM((bpw,), jnp.int32),
            pltpu.VMEM((NI,), jnp.int32),
            pltpu.VMEM((NI,), jnp.float32),
            pltpu.VMEM((NI,), jnp.float32),
            pltpu.VMEM((bpw,), jnp.int32),
            pltpu.VMEM((bpw,), jnp.float32),
            pltpu.VMEM((bpw,), jnp.float32),
            pltpu.SemaphoreType.DMA,
        ],
    )
    def k(stu_hbm, item_hbm, theta_hbm, a_hbm, b_hbm, map_hbm, out_hbm,
          stu_v, item_v, map_v, a_v, b_v, flat_v, tk_v, out_v, sem):
        wid = lax.axis_index("s") * NC + lax.axis_index("c")
        base = wid * bpw
        pltpu.sync_copy(stu_hbm.at[pl.ds(base, bpw)], stu_v)
        pltpu.sync_copy(item_hbm.at[pl.ds(base, bpw)], item_v)
        pltpu.sync_copy(map_hbm, map_v)
        pltpu.sync_copy(a_hbm, a_v)
        pltpu.sync_copy(b_hbm, b_v)
        for i in range(nv):
            item = item_v[pl.ds(i * L, L)]
            stu = stu_v[pl.ds(i * L, L)]
            skill = plsc.load_gather(map_v, [item])
            flat_v[pl.ds(i * L, L)] = stu * K + skill
        pltpu.async_copy(theta_hbm.at[flat_v], tk_v, sem).wait()
        for i in range(nv):
            item = item_v[pl.ds(i * L, L)]
            a_raw = plsc.load_gather(a_v, [item])
            b = plsc.load_gather(b_v, [item])
            tk = tk_v[pl.ds(i * L, L)]
            out_v[pl.ds(i * L, L)] = _softplus(a_raw) * (tk - b)
        pltpu.sync_copy(out_v, out_hbm.at[pl.ds(base, bpw)])

    return k


def kernel(stu_idx, item_idx, theta, a_w, b_w, item_skill_map):
    B = stu_idx.shape[0]
    ns, K = theta.shape
    NI = item_skill_map.shape[0]
    info = plsc.get_sparse_core_info()
    NC, NS = info.num_cores, info.num_subcores
    NW = NC * NS
    k = _make_sc_kernel(B, K, NI, NW, NC)
    return k(stu_idx, item_idx, theta.reshape(-1),
             a_w.reshape(-1), b_w.reshape(-1), item_skill_map)


# speedup vs baseline: 1.0004x; 1.0004x over previous
import functools

import jax
import jax.numpy as jnp
from jax import lax
from jax.experimental import pallas as pl
from jax.experimental.pallas import tpu as pltpu
from jax.experimental.pallas import tpu_sc as plsc

L = 16


def _softplus(x):
    t = jnp.exp(-jnp.abs(x))
    s = t / (t + 2.0)
    s2 = s * s
    atanh_s = s * (1.0 + s2 * (1.0 / 3.0 + s2 * (0.2 + s2 * (1.0 / 7.0 + s2 * (1.0 / 9.0)))))
    return jnp.maximum(x, 0.0) + 2.0 * atanh_s


def _make_sc_kernel(B, K, NI, NW, NC):
    bpw = B // NW
    nv = bpw // L
    mesh = plsc.VectorSubcoreMesh(core_axis_name="c", subcore_axis_name="s")

    @functools.partial(
        pl.kernel,
        mesh=mesh,
        compiler_params=pltpu.CompilerParams(
            needs_layout_passes=False, use_tc_tiling_on_sc=False),
        out_type=jax.ShapeDtypeStruct((B,), jnp.float32),
        scratch_types=[
            pltpu.VMEM((bpw,), jnp.int32),
            pltpu.VMEM((bpw,), jnp.int32),
            pltpu.VMEM((NI,), jnp.int32),
            pltpu.VMEM((NI,), jnp.float32),
            pltpu.VMEM((NI,), jnp.float32),
            pltpu.VMEM((bpw,), jnp.int32),
            pltpu.VMEM((bpw,), jnp.float32),
            pltpu.VMEM((bpw,), jnp.float32),
            pltpu.SemaphoreType.DMA,
        ],
    )
    def k(stu_hbm, item_hbm, theta_hbm, a_hbm, b_hbm, map_hbm, out_hbm,
          stu_v, item_v, map_v, a_v, b_v, flat_v, tk_v, out_v, sem):
        wid = lax.axis_index("s") * NC + lax.axis_index("c")
        base = wid * bpw
        pltpu.sync_copy(stu_hbm.at[pl.ds(base, bpw)], stu_v)
        pltpu.sync_copy(item_hbm.at[pl.ds(base, bpw)], item_v)
        pltpu.sync_copy(map_hbm, map_v)
        pltpu.sync_copy(a_hbm, a_v)
        pltpu.sync_copy(b_hbm, b_v)
        for i in range(nv):
            item = item_v[pl.ds(i * L, L)]
            stu = stu_v[pl.ds(i * L, L)]
            skill = plsc.load_gather(map_v, [item])
            flat_v[pl.ds(i * L, L)] = stu * K + skill
        pltpu.async_copy(theta_hbm.at[flat_v], tk_v, sem).wait()
        for i in range(nv):
            item = item_v[pl.ds(i * L, L)]
            a_raw = plsc.load_gather(a_v, [item])
            b = plsc.load_gather(b_v, [item])
            tk = tk_v[pl.ds(i * L, L)]
            out_v[pl.ds(i * L, L)] = _softplus(a_raw) * (tk - b)
        pltpu.sync_copy(out_v, out_hbm.at[pl.ds(base, bpw)])

    return k


def kernel(stu_idx, item_idx, theta, a_w, b_w, item_skill_map):
    B = stu_idx.shape[0]
    _, K = theta.shape
    NI = item_skill_map.shape[0]
    info = plsc.get_sparse_core_info()
    NC, NS = info.num_cores, info.num_subcores
    NW = NC * NS
    k = _make_sc_kernel(B, K, NI, NW, NC)
    return k(stu_idx, item_idx, theta.reshape(-1),
             a_w.reshape(-1), b_w.reshape(-1), item_skill_map)


# speedup vs baseline: 9.5937x; 9.5899x over previous
import functools

import jax
import jax.numpy as jnp
from jax import lax
from jax.experimental import pallas as pl
from jax.experimental.pallas import tpu as pltpu
from jax.experimental.pallas import tpu_sc as plsc

L = 16
WAVE = 32
SUB = 8
LANE = 128


def _softplus(x):
    t = jnp.exp(-jnp.abs(x))
    s = t / (t + 2.0)
    s2 = s * s
    atanh_s = s * (1.0 + s2 * (1.0 / 3.0 + s2 * (0.2 + s2 * (1.0 / 7.0 + s2 * (1.0 / 9.0)))))
    return jnp.maximum(x, 0.0) + 2.0 * atanh_s


def _make_sc_kernel(B, K, NI, NW, NC):
    bpw = B // NW
    nv = bpw // L
    nwave = bpw // WAVE
    mesh = plsc.VectorSubcoreMesh(core_axis_name="c", subcore_axis_name="s")

    @functools.partial(
        pl.kernel,
        mesh=mesh,
        compiler_params=pltpu.CompilerParams(
            needs_layout_passes=False, use_tc_tiling_on_sc=True),
        out_type=jax.ShapeDtypeStruct((B,), jnp.float32),
        scratch_types=[
            pltpu.VMEM((bpw,), jnp.int32),
            pltpu.VMEM((bpw,), jnp.int32),
            pltpu.VMEM((NI,), jnp.int32),
            pltpu.VMEM((NI,), jnp.float32),
            pltpu.VMEM((NI,), jnp.float32),
            pltpu.VMEM((bpw,), jnp.int32),
            pltpu.VMEM((bpw,), jnp.int32),
            pltpu.VMEM((bpw,), jnp.int32),
            pltpu.VMEM((SUB, WAVE * LANE), jnp.float32),
            pltpu.VMEM((SUB, WAVE * LANE), jnp.float32),
            pltpu.VMEM((bpw,), jnp.float32),
            pltpu.SemaphoreType.DMA,
            pltpu.SemaphoreType.DMA,
        ],
    )
    def k(stu_hbm, item_hbm, theta_t_hbm, a_hbm, b_hbm, map_hbm, out_hbm,
          stu_v, item_v, map_v, a_v, b_v, skill_v, row8_v, col_v,
          gbuf0, gbuf1, out_v, sem0, sem1):
        wid = lax.axis_index("s") * NC + lax.axis_index("c")
        qbase = wid * bpw
        pltpu.sync_copy(stu_hbm.at[pl.ds(qbase, bpw)], stu_v)
        pltpu.sync_copy(item_hbm.at[pl.ds(qbase, bpw)], item_v)
        pltpu.sync_copy(map_hbm, map_v)
        pltpu.sync_copy(a_hbm, a_v)
        pltpu.sync_copy(b_hbm, b_v)
        for i in range(nv):
            sl = pl.ds(i * L, L)
            item = item_v[sl]
            stu = stu_v[sl]
            sk = plsc.load_gather(map_v, [item])
            skill_v[sl] = sk
            row8_v[sl] = lax.bitwise_and(sk, ~(SUB - 1))
            col_v[sl] = lax.bitwise_and(stu, ~(LANE - 1))
        gbufs = (gbuf0, gbuf1)
        sems = (sem0, sem1)

        def fire(w, gbuf, sem):
            for vi in range(WAVE // L):
                r8v = row8_v[pl.ds(w * WAVE + vi * L, L)]
                cbv = col_v[pl.ds(w * WAVE + vi * L, L)]
                for j in range(L):
                    r8 = pl.multiple_of(r8v[j], SUB)
                    cb = pl.multiple_of(cbv[j], LANE)
                    pltpu.async_copy(
                        theta_t_hbm.at[pl.ds(r8, SUB), pl.ds(cb, LANE)],
                        gbuf.at[:, pl.ds((vi * L + j) * LANE, LANE)],
                        sem)

        def drain(gbuf, sem):
            pltpu.make_async_copy(
                theta_t_hbm.at[pl.ds(0, SUB), pl.ds(0, WAVE * LANE)],
                gbuf, sem).wait()

        def extract(w, gbuf):
            for v in range(WAVE // L):
                i = (w * WAVE) // L + v
                sl = pl.ds(i * L, L)
                stu = stu_v[sl]
                sk = skill_v[sl]
                item = item_v[sl]
                idx0 = lax.bitwise_and(sk, SUB - 1)
                idx1 = (L * v + lax.iota(jnp.int32, L)) * LANE + lax.bitwise_and(stu, LANE - 1)
                tk = plsc.load_gather(gbuf, [idx0, idx1])
                a_raw = plsc.load_gather(a_v, [item])
                b = plsc.load_gather(b_v, [item])
                out_v[sl] = _softplus(a_raw) * (tk - b)

        fire(0, gbufs[0], sems[0])
        for w in range(1, nwave):
            fire(w, gbufs[w % 2], sems[w % 2])
            drain(gbufs[(w - 1) % 2], sems[(w - 1) % 2])
            extract(w - 1, gbufs[(w - 1) % 2])
        drain(gbufs[(nwave - 1) % 2], sems[(nwave - 1) % 2])
        extract(nwave - 1, gbufs[(nwave - 1) % 2])
        pltpu.sync_copy(out_v, out_hbm.at[pl.ds(qbase, bpw)])

    return k


def kernel(stu_idx, item_idx, theta, a_w, b_w, item_skill_map):
    B = stu_idx.shape[0]
    _, K = theta.shape
    NI = item_skill_map.shape[0]
    info = plsc.get_sparse_core_info()
    NC, NS = info.num_cores, info.num_subcores
    NW = NC * NS
    k = _make_sc_kernel(B, K, NI, NW, NC)
    return k(stu_idx, item_idx, theta.T,
             a_w.reshape(-1), b_w.reshape(-1), item_skill_map)


# speedup vs baseline: 10.0385x; 1.0464x over previous
import functools

import jax
import jax.numpy as jnp
from jax import lax
from jax.experimental import pallas as pl
from jax.experimental.pallas import tpu as pltpu
from jax.experimental.pallas import tpu_sc as plsc

L = 16
WAVE = 32
SUB = 8
LANE = 128


def _softplus(x):
    t = jnp.exp(-jnp.abs(x))
    s = t / (t + 2.0)
    s2 = s * s
    atanh_s = s * (1.0 + s2 * (1.0 / 3.0 + s2 * (0.2 + s2 * (1.0 / 7.0 + s2 * (1.0 / 9.0)))))
    return jnp.maximum(x, 0.0) + 2.0 * atanh_s


def _make_sc_kernel(B, K, NI, NW, NC):
    bpw = B // NW
    nv = bpw // L
    nwave = bpw // WAVE
    mesh = plsc.VectorSubcoreMesh(core_axis_name="c", subcore_axis_name="s")

    @functools.partial(
        pl.kernel,
        mesh=mesh,
        compiler_params=pltpu.CompilerParams(
            needs_layout_passes=False, use_tc_tiling_on_sc=True),
        out_type=jax.ShapeDtypeStruct((B,), jnp.float32),
        scratch_types=[
            pltpu.VMEM((bpw,), jnp.int32),
            pltpu.VMEM((bpw,), jnp.int32),
            pltpu.VMEM((NI,), jnp.int32),
            pltpu.VMEM((NI,), jnp.float32),
            pltpu.VMEM((NI,), jnp.float32),
            pltpu.VMEM((bpw,), jnp.int32),
            pltpu.VMEM((bpw,), jnp.int32),
            pltpu.VMEM((bpw,), jnp.int32),
            pltpu.VMEM((SUB, WAVE * LANE), jnp.float32),
            pltpu.VMEM((SUB, WAVE * LANE), jnp.float32),
            pltpu.VMEM((bpw,), jnp.float32),
            pltpu.SemaphoreType.DMA,
            pltpu.SemaphoreType.DMA,
            pltpu.SemaphoreType.DMA,
        ],
    )
    def k(stu_hbm, item_hbm, theta_t_hbm, a_hbm, b_hbm, map_hbm, out_hbm,
          stu_v, item_v, map_v, a_v, b_v, skill_v, row8_v, col_v,
          gbuf0, gbuf1, out_v, sem0, sem1, psem):
        wid = lax.axis_index("s") * NC + lax.axis_index("c")
        qbase = wid * bpw
        d0 = pltpu.async_copy(stu_hbm.at[pl.ds(qbase, bpw)], stu_v, psem)
        d1 = pltpu.async_copy(item_hbm.at[pl.ds(qbase, bpw)], item_v, psem)
        d2 = pltpu.async_copy(map_hbm, map_v, psem)
        d3 = pltpu.async_copy(a_hbm, a_v, psem)
        d4 = pltpu.async_copy(b_hbm, b_v, psem)
        d0.wait()
        d1.wait()
        d2.wait()
        for i in range(nv):
            sl = pl.ds(i * L, L)
            item = item_v[sl]
            stu = stu_v[sl]
            sk = plsc.load_gather(map_v, [item])
            skill_v[sl] = sk
            row8_v[sl] = lax.bitwise_and(sk, ~(SUB - 1))
            col_v[sl] = lax.bitwise_and(stu, ~(LANE - 1))
        gbufs = (gbuf0, gbuf1)
        sems = (sem0, sem1)

        def fire(w, gbuf, sem):
            for vi in range(WAVE // L):
                r8v = row8_v[pl.ds(w * WAVE + vi * L, L)]
                cbv = col_v[pl.ds(w * WAVE + vi * L, L)]
                for j in range(L):
                    r8 = pl.multiple_of(r8v[j], SUB)
                    cb = pl.multiple_of(cbv[j], LANE)
                    pltpu.async_copy(
                        theta_t_hbm.at[pl.ds(r8, SUB), pl.ds(cb, LANE)],
                        gbuf.at[:, pl.ds((vi * L + j) * LANE, LANE)],
                        sem)

        def drain(gbuf, sem):
            pltpu.make_async_copy(
                theta_t_hbm.at[pl.ds(0, SUB), pl.ds(0, WAVE * LANE)],
                gbuf, sem).wait()

        def extract(w, gbuf):
            for v in range(WAVE // L):
                i = (w * WAVE) // L + v
                sl = pl.ds(i * L, L)
                stu = stu_v[sl]
                sk = skill_v[sl]
                item = item_v[sl]
                idx0 = lax.bitwise_and(sk, SUB - 1)
                idx1 = (L * v + lax.iota(jnp.int32, L)) * LANE + lax.bitwise_and(stu, LANE - 1)
                tk = plsc.load_gather(gbuf, [idx0, idx1])
                a_raw = plsc.load_gather(a_v, [item])
                b = plsc.load_gather(b_v, [item])
                out_v[sl] = _softplus(a_raw) * (tk - b)

        fire(0, gbufs[0], sems[0])
        d3.wait()
        d4.wait()
        for w in range(1, nwave):
            fire(w, gbufs[w % 2], sems[w % 2])
            drain(gbufs[(w - 1) % 2], sems[(w - 1) % 2])
            extract(w - 1, gbufs[(w - 1) % 2])
        drain(gbufs[(nwave - 1) % 2], sems[(nwave - 1) % 2])
        extract(nwave - 1, gbufs[(nwave - 1) % 2])
        pltpu.sync_copy(out_v, out_hbm.at[pl.ds(qbase, bpw)])

    return k


def kernel(stu_idx, item_idx, theta, a_w, b_w, item_skill_map):
    B = stu_idx.shape[0]
    _, K = theta.shape
    NI = item_skill_map.shape[0]
    info = plsc.get_sparse_core_info()
    NC, NS = info.num_cores, info.num_subcores
    NW = NC * NS
    k = _make_sc_kernel(B, K, NI, NW, NC)
    return k(stu_idx, item_idx, theta.T,
             a_w.reshape(-1), b_w.reshape(-1), item_skill_map)


# speedup vs baseline: 11.3165x; 1.1273x over previous
import functools

import jax
import jax.numpy as jnp
from jax import lax
from jax.experimental import pallas as pl
from jax.experimental.pallas import tpu as pltpu
from jax.experimental.pallas import tpu_sc as plsc

L = 16
WAVE = 32
SUB = 8
LANE = 128


def _softplus(x):
    t = jnp.exp(-jnp.abs(x))
    s = t / (t + 2.0)
    s2 = s * s
    atanh_s = s * (1.0 + s2 * (1.0 / 3.0 + s2 * (0.2 + s2 * (1.0 / 7.0 + s2 * (1.0 / 9.0)))))
    return jnp.maximum(x, 0.0) + 2.0 * atanh_s


def _make_sc_kernel(B, K, NI, NW, NC):
    bpw = B // NW
    nv = bpw // L
    nwave = bpw // WAVE
    mesh = plsc.VectorSubcoreMesh(core_axis_name="c", subcore_axis_name="s")

    @functools.partial(
        pl.kernel,
        mesh=mesh,
        compiler_params=pltpu.CompilerParams(
            needs_layout_passes=False, use_tc_tiling_on_sc=True),
        out_type=jax.ShapeDtypeStruct((B,), jnp.float32),
        scratch_types=[
            pltpu.VMEM((bpw,), jnp.int32),
            pltpu.VMEM((bpw,), jnp.int32),
            pltpu.VMEM((NI,), jnp.int32),
            pltpu.VMEM((NI,), jnp.float32),
            pltpu.VMEM((NI,), jnp.float32),
            pltpu.VMEM((bpw,), jnp.int32),
            pltpu.VMEM((bpw,), jnp.int32),
            pltpu.VMEM((bpw,), jnp.int32),
            pltpu.VMEM((SUB, WAVE * LANE), jnp.float32),
            pltpu.VMEM((SUB, WAVE * LANE), jnp.float32),
            pltpu.VMEM((bpw,), jnp.float32),
            pltpu.SemaphoreType.DMA,
            pltpu.SemaphoreType.DMA,
            pltpu.SemaphoreType.DMA,
        ],
    )
    def k(stu_hbm, item_hbm, theta_t_hbm, a_hbm, b_hbm, map_hbm, out_hbm,
          stu_v, item_v, map_v, a_v, b_v, skill_v, row8_v, col_v,
          gbuf0, gbuf1, out_v, sem0, sem1, psem):
        wid = lax.axis_index("s") * NC + lax.axis_index("c")
        qbase = wid * bpw
        d0 = pltpu.async_copy(stu_hbm.at[pl.ds(qbase, bpw)], stu_v, psem)
        d1 = pltpu.async_copy(item_hbm.at[pl.ds(qbase, bpw)], item_v, psem)
        d2 = pltpu.async_copy(map_hbm, map_v, psem)
        d3 = pltpu.async_copy(a_hbm, a_v, psem)
        d4 = pltpu.async_copy(b_hbm, b_v, psem)
        d0.wait()
        d1.wait()
        d2.wait()
        for i in range(nv):
            sl = pl.ds(i * L, L)
            item = item_v[sl]
            stu = stu_v[sl]
            sk = plsc.load_gather(map_v, [item])
            skill_v[sl] = sk
            row8_v[sl] = lax.bitwise_and(sk, ~(SUB - 1))
            col_v[sl] = lax.bitwise_and(stu, ~(LANE - 1))
        def fire(q0, gbuf, sem):
            for vi in range(WAVE // L):
                r8v = row8_v[pl.ds(q0 + vi * L, L)]
                cbv = col_v[pl.ds(q0 + vi * L, L)]
                for j in range(L):
                    r8 = pl.multiple_of(r8v[j], SUB)
                    cb = pl.multiple_of(cbv[j], LANE)
                    pltpu.async_copy(
                        theta_t_hbm.at[pl.ds(r8, SUB), pl.ds(cb, LANE)],
                        gbuf.at[:, pl.ds((vi * L + j) * LANE, LANE)],
                        sem)

        def drain(gbuf, sem):
            pltpu.make_async_copy(
                theta_t_hbm.at[pl.ds(0, SUB), pl.ds(0, WAVE * LANE)],
                gbuf, sem).wait()

        def extract(q0, gbuf):
            for vi in range(WAVE // L):
                sl = pl.ds(q0 + vi * L, L)
                stu = stu_v[sl]
                sk = skill_v[sl]
                item = item_v[sl]
                idx0 = lax.bitwise_and(sk, SUB - 1)
                idx1 = (L * vi + lax.iota(jnp.int32, L)) * LANE + lax.bitwise_and(stu, LANE - 1)
                tk = plsc.load_gather(gbuf, [idx0, idx1])
                a_raw = plsc.load_gather(a_v, [item])
                b = plsc.load_gather(b_v, [item])
                out_v[sl] = _softplus(a_raw) * (tk - b)

        fire(0, gbuf0, sem0)
        d3.wait()
        d4.wait()

        def pair(p, carry):
            q1 = p * 2 * WAVE + WAVE
            fire(q1, gbuf1, sem1)
            drain(gbuf0, sem0)
            extract(q1 - WAVE, gbuf0)
            fire(q1 + WAVE, gbuf0, sem0)
            drain(gbuf1, sem1)
            extract(q1, gbuf1)
            return carry

        lax.fori_loop(0, nwave // 2 - 1, pair, 0)
        qlast = (nwave - 1) * WAVE
        fire(qlast, gbuf1, sem1)
        drain(gbuf0, sem0)
        extract(qlast - WAVE, gbuf0)
        drain(gbuf1, sem1)
        extract(qlast, gbuf1)
        pltpu.sync_copy(out_v, out_hbm.at[pl.ds(qbase, bpw)])

    return k


def kernel(stu_idx, item_idx, theta, a_w, b_w, item_skill_map):
    B = stu_idx.shape[0]
    _, K = theta.shape
    NI = item_skill_map.shape[0]
    info = plsc.get_sparse_core_info()
    NC, NS = info.num_cores, info.num_subcores
    NW = NC * NS
    k = _make_sc_kernel(B, K, NI, NW, NC)
    return k(stu_idx, item_idx, theta.T,
             a_w.reshape(-1), b_w.reshape(-1), item_skill_map)


# speedup vs baseline: 11.5503x; 1.0207x over previous
import functools

import jax
import jax.numpy as jnp
from jax import lax
from jax.experimental import pallas as pl
from jax.experimental.pallas import tpu as pltpu
from jax.experimental.pallas import tpu_sc as plsc

L = 16
WAVE = 32
SUB = 8
LANE = 128


def _softplus(x):
    t = jnp.exp(-jnp.abs(x))
    s = t / (t + 2.0)
    s2 = s * s
    atanh_s = s * (1.0 + s2 * (1.0 / 3.0 + s2 * (0.2 + s2 * (1.0 / 7.0 + s2 * (1.0 / 9.0)))))
    return jnp.maximum(x, 0.0) + 2.0 * atanh_s


def _make_sc_kernel(B, K, NI, NW, NC):
    bpw = B // NW
    nv = bpw // L
    nwave = bpw // WAVE
    mesh = plsc.VectorSubcoreMesh(core_axis_name="c", subcore_axis_name="s")

    @functools.partial(
        pl.kernel,
        mesh=mesh,
        compiler_params=pltpu.CompilerParams(
            needs_layout_passes=False, use_tc_tiling_on_sc=True),
        out_type=jax.ShapeDtypeStruct((B,), jnp.float32),
        scratch_types=[
            pltpu.VMEM((bpw,), jnp.int32),
            pltpu.VMEM((bpw,), jnp.int32),
            pltpu.VMEM((NI,), jnp.int32),
            pltpu.VMEM((NI,), jnp.float32),
            pltpu.VMEM((NI,), jnp.float32),
            pltpu.VMEM((bpw,), jnp.int32),
            pltpu.VMEM((bpw,), jnp.int32),
            pltpu.VMEM((bpw,), jnp.int32),
            pltpu.VMEM((SUB, WAVE * LANE), jnp.float32),
            pltpu.VMEM((SUB, WAVE * LANE), jnp.float32),
            pltpu.VMEM((SUB, WAVE * LANE), jnp.float32),
            pltpu.VMEM((bpw,), jnp.float32),
            pltpu.SemaphoreType.DMA,
            pltpu.SemaphoreType.DMA,
            pltpu.SemaphoreType.DMA,
            pltpu.SemaphoreType.DMA,
        ],
    )
    def k(stu_hbm, item_hbm, theta_t_hbm, a_hbm, b_hbm, map_hbm, out_hbm,
          stu_v, item_v, map_v, a_v, b_v, skill_v, row8_v, col_v,
          gbuf0, gbuf1, gbuf2, out_v, sem0, sem1, sem2, psem):
        wid = lax.axis_index("s") * NC + lax.axis_index("c")
        qbase = wid * bpw
        d0 = pltpu.async_copy(stu_hbm.at[pl.ds(qbase, bpw)], stu_v, psem)
        d1 = pltpu.async_copy(item_hbm.at[pl.ds(qbase, bpw)], item_v, psem)
        d2 = pltpu.async_copy(map_hbm, map_v, psem)
        d3 = pltpu.async_copy(a_hbm, a_v, psem)
        d4 = pltpu.async_copy(b_hbm, b_v, psem)
        d0.wait()
        d1.wait()
        d2.wait()
        for i in range(nv):
            sl = pl.ds(i * L, L)
            item = item_v[sl]
            stu = stu_v[sl]
            sk = plsc.load_gather(map_v, [item])
            skill_v[sl] = sk
            row8_v[sl] = lax.bitwise_and(sk, ~(SUB - 1))
            col_v[sl] = lax.bitwise_and(stu, ~(LANE - 1))
        def fire(q0, gbuf, sem):
            for vi in range(WAVE // L):
                r8v = row8_v[pl.ds(q0 + vi * L, L)]
                cbv = col_v[pl.ds(q0 + vi * L, L)]
                for j in range(L):
                    r8 = pl.multiple_of(r8v[j], SUB)
                    cb = pl.multiple_of(cbv[j], LANE)
                    pltpu.async_copy(
                        theta_t_hbm.at[pl.ds(r8, SUB), pl.ds(cb, LANE)],
                        gbuf.at[:, pl.ds((vi * L + j) * LANE, LANE)],
                        sem)

        def drain(gbuf, sem):
            pltpu.make_async_copy(
                theta_t_hbm.at[pl.ds(0, SUB), pl.ds(0, WAVE * LANE)],
                gbuf, sem).wait()

        def extract(q0, gbuf):
            for vi in range(WAVE // L):
                sl = pl.ds(q0 + vi * L, L)
                stu = stu_v[sl]
                sk = skill_v[sl]
                item = item_v[sl]
                idx0 = lax.bitwise_and(sk, SUB - 1)
                idx1 = (L * vi + lax.iota(jnp.int32, L)) * LANE + lax.bitwise_and(stu, LANE - 1)
                tk = plsc.load_gather(gbuf, [idx0, idx1])
                a_raw = plsc.load_gather(a_v, [item])
                b = plsc.load_gather(b_v, [item])
                out_v[sl] = _softplus(a_raw) * (tk - b)

        fire(0, gbuf0, sem0)
        fire(WAVE, gbuf1, sem1)
        d3.wait()
        d4.wait()

        def triple(h, carry):
            base = h * 3 * WAVE
            fire(base + 2 * WAVE, gbuf2, sem2)
            drain(gbuf0, sem0)
            extract(base, gbuf0)
            fire(base + 3 * WAVE, gbuf0, sem0)
            drain(gbuf1, sem1)
            extract(base + WAVE, gbuf1)
            fire(base + 4 * WAVE, gbuf1, sem1)
            drain(gbuf2, sem2)
            extract(base + 2 * WAVE, gbuf2)
            return carry

        nloop = (nwave - 4) // 3
        lax.fori_loop(0, nloop, triple, 0)
        tb = nloop * 3 * WAVE
        fire(tb + 2 * WAVE, gbuf2, sem2)
        drain(gbuf0, sem0)
        extract(tb, gbuf0)
        fire(tb + 3 * WAVE, gbuf0, sem0)
        drain(gbuf1, sem1)
        extract(tb + WAVE, gbuf1)
        drain(gbuf2, sem2)
        extract(tb + 2 * WAVE, gbuf2)
        drain(gbuf0, sem0)
        extract(tb + 3 * WAVE, gbuf0)
        pltpu.sync_copy(out_v, out_hbm.at[pl.ds(qbase, bpw)])

    return k


def kernel(stu_idx, item_idx, theta, a_w, b_w, item_skill_map):
    B = stu_idx.shape[0]
    _, K = theta.shape
    NI = item_skill_map.shape[0]
    info = plsc.get_sparse_core_info()
    NC, NS = info.num_cores, info.num_subcores
    NW = NC * NS
    k = _make_sc_kernel(B, K, NI, NW, NC)
    return k(stu_idx, item_idx, theta.T,
             a_w.reshape(-1), b_w.reshape(-1), item_skill_map)
